# baseline (speedup 1.0000x reference)
import functools

import jax
import jax.numpy as jnp
from jax import lax
from jax.experimental import pallas as pl
from jax.experimental.pallas import tpu as pltpu
from jax.experimental.pallas import tpu_sc as plsc

N = 16 * 1024 * 1024
L = 16
CHUNK = 32768
VECS = CHUNK // L


def _make_sc_call():
  info = plsc.get_sparse_core_info()
  nw = info.num_cores * info.num_subcores
  per_w = N // nw
  n_chunks = per_w // CHUNK
  mesh = plsc.VectorSubcoreMesh(core_axis_name="c", subcore_axis_name="s")

  @functools.partial(
      pl.kernel,
      out_type=jax.ShapeDtypeStruct((nw * L,), jnp.int32),
      mesh=mesh,
      compiler_params=pltpu.CompilerParams(needs_layout_passes=False),
      scratch_types=[
          pltpu.VMEM((CHUNK,), jnp.float32),
          pltpu.VMEM((CHUNK,), jnp.int32),
          pltpu.VMEM((L,), jnp.int32),
      ],
  )
  def sc_kernel(x_hbm, out_hbm, xbuf, idxbuf, flag_v):
    wid = lax.axis_index("s") * info.num_cores + lax.axis_index("c")
    base_w = wid * per_w
    lane = lax.iota(jnp.int32, L)
    ones = jnp.ones((L,), jnp.int32)
    zeros = jnp.zeros((L,), jnp.int32)

    ch_l = CHUNK // L
    lane_base = lane * ch_l

    def chunk_body(c, flag):
      base_c = base_w + c * CHUNK
      pltpu.sync_copy(x_hbm.at[pl.ds(base_c, CHUNK)], xbuf)

      def vec_body(t, carry):
        ptr, flag = carry
        v = plsc.load_gather(xbuf, [lane_base + t])
        m = v != 0.0
        idxs = base_c + lane_base + t
        pos = lane_base + ptr
        plsc.store_scatter(idxbuf, [pos], idxs, mask=m)
        d = plsc.load_gather(idxbuf, [pos], mask=m)
        flag = jnp.where(m & (d != idxs), zeros, flag)
        ptr = ptr + jnp.where(m, ones, zeros)
        return ptr, flag

      _, flag = lax.fori_loop(0, ch_l, vec_body, (zeros, flag))
      return flag

    flag = lax.fori_loop(0, n_chunks, chunk_body, ones)
    flag_v[...] = flag
    pltpu.sync_copy(flag_v, out_hbm.at[pl.ds(wid * L, L)])

  return sc_kernel


_sc_call = None


def kernel(x):
  global _sc_call
  if _sc_call is None:
    _sc_call = _make_sc_call()
  flags = _sc_call(x)
  return jnp.all(flags == 1)


# speedup vs baseline: 1.1761x; 1.1761x over previous
import functools

import jax
import jax.numpy as jnp
from jax import lax
from jax.experimental import pallas as pl
from jax.experimental.pallas import tpu as pltpu
from jax.experimental.pallas import tpu_sc as plsc

N = 16 * 1024 * 1024
L = 16
CHUNK = 16384
U = 8


def _make_sc_call():
  info = plsc.get_sparse_core_info()
  nw = info.num_cores * info.num_subcores
  per_w = N // nw
  n_pairs = per_w // (2 * CHUNK)
  vecs = CHUNK // L
  ch_l = CHUNK // L
  mesh = plsc.VectorSubcoreMesh(core_axis_name="c", subcore_axis_name="s")

  @functools.partial(
      pl.kernel,
      out_type=jax.ShapeDtypeStruct((nw * L,), jnp.int32),
      mesh=mesh,
      compiler_params=pltpu.CompilerParams(needs_layout_passes=False),
      scratch_types=[
          pltpu.VMEM((CHUNK,), jnp.float32),
          pltpu.VMEM((CHUNK,), jnp.float32),
          pltpu.VMEM((CHUNK,), jnp.int32),
          pltpu.VMEM((CHUNK,), jnp.int32),
          pltpu.VMEM((L,), jnp.int32),
          pltpu.SemaphoreType.DMA,
          pltpu.SemaphoreType.DMA,
      ],
  )
  def sc_kernel(x_hbm, out_hbm, xb0, xb1, dep, cor, flag_v, sem0, sem1):
    wid = lax.axis_index("s") * info.num_cores + lax.axis_index("c")
    base_w = wid * per_w
    lane = lax.iota(jnp.int32, L)
    ones = jnp.ones((L,), jnp.int32)
    zeros = jnp.zeros((L,), jnp.int32)
    chain_base = lane * ch_l

    def z_body(s, _):
      for u in range(U):
        t = s * U + u
        dep[pl.ds(t * L, L)] = zeros
        cor[pl.ds(t * L, L)] = zeros
      return 0

    lax.fori_loop(0, vecs // U, z_body, 0)

    def compact_and_check(xbuf, base_c, bad):
      def p1_body(s, ptr):
        for u in range(U):
          t = s * U + u
          v = xbuf[pl.ds(t * L, L)]
          m = v != 0.0
          idxs = (base_c + t * L) + lane
          pos = chain_base + ptr
          plsc.store_scatter(dep, [pos], idxs, mask=m)
          plsc.store_scatter(cor, [pos], idxs, mask=m)
          ptr = ptr + jnp.where(m, ones, zeros)
        return ptr

      lax.fori_loop(0, vecs // U, p1_body, zeros)

      def p2_body(s, bad):
        for u in range(U):
          t = s * U + u
          a = dep[pl.ds(t * L, L)]
          b = cor[pl.ds(t * L, L)]
          bad = bad | (a ^ b)
        return bad

      return lax.fori_loop(0, vecs // U, p2_body, bad)

    pltpu.async_copy(x_hbm.at[pl.ds(base_w, CHUNK)], xb0, sem0)

    def pair_body(p, bad):
      base_c0 = base_w + (2 * p) * CHUNK
      base_c1 = base_c0 + CHUNK
      pltpu.make_async_copy(x_hbm.at[pl.ds(0, CHUNK)], xb0, sem0).wait()
      pltpu.async_copy(x_hbm.at[pl.ds(base_c1, CHUNK)], xb1, sem1)
      bad = compact_and_check(xb0, base_c0, bad)
      pltpu.make_async_copy(x_hbm.at[pl.ds(0, CHUNK)], xb1, sem1).wait()
      nxt = jnp.minimum(base_c0 + 2 * CHUNK, N - CHUNK)
      pltpu.async_copy(x_hbm.at[pl.ds(nxt, CHUNK)], xb0, sem0)
      bad = compact_and_check(xb1, base_c1, bad)
      return bad

    bad = lax.fori_loop(0, n_pairs, pair_body, zeros)
    pltpu.make_async_copy(x_hbm.at[pl.ds(0, CHUNK)], xb0, sem0).wait()

    flag_v[...] = jnp.where(bad == 0, ones, zeros)
    pltpu.sync_copy(flag_v, out_hbm.at[pl.ds(wid * L, L)])

  return sc_kernel


_sc_call = None


def kernel(x):
  global _sc_call
  if _sc_call is None:
    _sc_call = _make_sc_call()
  flags = _sc_call(x)
  return jnp.all(flags == 1)


# speedup vs baseline: 2.3145x; 1.9679x over previous
import functools

import jax
import jax.numpy as jnp
from jax import lax
from jax.experimental import pallas as pl
from jax.experimental.pallas import tpu as pltpu
from jax.experimental.pallas import tpu_sc as plsc

N = 16 * 1024 * 1024
L = 16
CHUNK = 16384
UNROLL = 8


def _make_sc_call():
  info = plsc.get_sparse_core_info()
  nw = info.num_cores * info.num_subcores
  per_w = N // nw
  n_pairs = per_w // (2 * CHUNK)
  cap = CHUNK // L
  vecs = CHUNK // L
  mesh = plsc.VectorSubcoreMesh(core_axis_name="c", subcore_axis_name="s")

  @functools.partial(
      pl.kernel,
      out_type=jax.ShapeDtypeStruct((nw * L,), jnp.int32),
      mesh=mesh,
      compiler_params=pltpu.CompilerParams(needs_layout_passes=False),
      scratch_types=[
          pltpu.VMEM((CHUNK,), jnp.float32),
          pltpu.VMEM((CHUNK,), jnp.float32),
          pltpu.VMEM((CHUNK,), jnp.int32),
          pltpu.VMEM((L,), jnp.int32),
          pltpu.SemaphoreType.DMA,
          pltpu.SemaphoreType.DMA,
      ],
  )
  def sc_kernel(x_hbm, out_hbm, xb0, xb1, idxb, flag_v, sem0, sem1):
    wid = lax.axis_index("s") * info.num_cores + lax.axis_index("c")
    base_w = wid * per_w
    lane = lax.iota(jnp.int32, L)
    ones = jnp.ones((L,), jnp.int32)
    zeros = jnp.zeros((L,), jnp.int32)
    chain_base = lane * cap

    def compact_and_check(xbuf, base_c, bad):
      def body(t, carry):
        ptr, bad = carry
        v = plsc.bitcast(xbuf[pl.ds(t * L, L)], jnp.int32)
        nzb = v & 0x7FFFFFFF
        m = nzb != 0
        mi = jnp.minimum(nzb, 1)
        idxs = (base_c + t * L) + lane
        pos = chain_base + ptr
        plsc.store_scatter(idxb, [pos], idxs, mask=m)
        d = plsc.load_gather(idxb, [pos], mask=m)
        bad = bad | jnp.where(m, d ^ idxs, zeros)
        return ptr + mi, bad

      return plsc.parallel_loop(
          0, vecs, unroll=UNROLL, carry=(zeros, bad))(body)[1]

    pltpu.async_copy(x_hbm.at[pl.ds(base_w, CHUNK)], xb0, sem0)

    def pair_body(p, bad):
      base_c0 = base_w + (2 * p) * CHUNK
      base_c1 = base_c0 + CHUNK
      pltpu.make_async_copy(x_hbm.at[pl.ds(0, CHUNK)], xb0, sem0).wait()
      pltpu.async_copy(x_hbm.at[pl.ds(base_c1, CHUNK)], xb1, sem1)
      bad = compact_and_check(xb0, base_c0, bad)
      pltpu.make_async_copy(x_hbm.at[pl.ds(0, CHUNK)], xb1, sem1).wait()
      nxt = jnp.minimum(base_c0 + 2 * CHUNK, N - CHUNK)
      pltpu.async_copy(x_hbm.at[pl.ds(nxt, CHUNK)], xb0, sem0)
      bad = compact_and_check(xb1, base_c1, bad)
      return bad

    bad = lax.fori_loop(0, n_pairs, pair_body, zeros)
    pltpu.make_async_copy(x_hbm.at[pl.ds(0, CHUNK)], xb0, sem0).wait()

    flag_v[...] = jnp.where(bad == 0, ones, zeros)
    pltpu.sync_copy(flag_v, out_hbm.at[pl.ds(wid * L, L)])

  return sc_kernel


_sc_call = None


def kernel(x):
  global _sc_call
  if _sc_call is None:
    _sc_call = _make_sc_call()
  flags = _sc_call(x)
  return jnp.all(flags == 1)
